# baseline (device time: 201507 ns/iter reference)
import jax
import jax.numpy as jnp
from jax import lax
from jax.experimental import pallas as pl
from jax.experimental.pallas import tpu as pltpu

N_DEV = 8
SQ_BLK = 256
D_MODEL = 1024
H_PER = 8
DH = 128
WIN = 512
K_ROWS = 2176
SCALE = 0.08838834764831843
NEG = -1e9


def _body(x_ref, wq_ref, k_ref, v_ref, wo_ref, out_ref,
          comm_x, acc,
          x_send_sems, x_recv_sems, acc_send_sems, acc_recv_sems,
          out_send_sem, out_recv_sem):
    my = lax.axis_index("i")
    right = lax.rem(my + 1, N_DEV)
    left = lax.rem(my + N_DEV - 1, N_DEV)

    barrier = pltpu.get_barrier_semaphore()
    for nbr in (left, right):
        pl.semaphore_signal(barrier, inc=1, device_id=(nbr,),
                            device_id_type=pl.DeviceIdType.MESH)
    pl.semaphore_wait(barrier, 2)

    def contribution(x_j, j):
        q = jnp.dot(x_j, wq_ref[...], preferred_element_type=jnp.float32)
        start = jnp.maximum(256 * j - 128, 0)
        rows = lax.broadcasted_iota(jnp.int32, (SQ_BLK, WIN), 0)
        cols = lax.broadcasted_iota(jnp.int32, (SQ_BLK, WIN), 1)
        mask = jnp.abs((rows + 256 * j) - (cols + start)) <= 128
        ctxs = []
        for hh in range(H_PER):
            qh = q[:, hh * DH:(hh + 1) * DH]
            kh = k_ref[hh, pl.ds(start, WIN), :]
            s = lax.dot_general(qh, kh, (((1,), (1,)), ((), ())),
                                preferred_element_type=jnp.float32)
            s = jnp.where(mask, s * SCALE, NEG)
            m = jnp.max(s, axis=-1, keepdims=True)
            w = jnp.exp(s - m)
            w = w / jnp.sum(w, axis=-1, keepdims=True)
            vh = v_ref[hh, pl.ds(start, WIN), :]
            ctxs.append(jnp.dot(w, vh, preferred_element_type=jnp.float32))
        ctx = jnp.concatenate(ctxs, axis=1)
        return jnp.dot(ctx, wo_ref[...], preferred_element_type=jnp.float32)

    send_waits = []
    for h in range(N_DEV):
        j = lax.rem(my - h + N_DEV, N_DEV)

        if h == 0:
            x_src = x_ref
        else:
            pltpu.make_async_remote_copy(
                src_ref=comm_x.at[h], dst_ref=comm_x.at[h],
                send_sem=x_send_sems.at[h], recv_sem=x_recv_sems.at[h],
                device_id=(left,), device_id_type=pl.DeviceIdType.MESH,
            ).wait_recv()
            x_src = comm_x.at[h]

        if h < N_DEV - 1:
            x_fwd = pltpu.make_async_remote_copy(
                src_ref=x_src, dst_ref=comm_x.at[h + 1],
                send_sem=x_send_sems.at[h], recv_sem=x_recv_sems.at[h + 1],
                device_id=(right,), device_id_type=pl.DeviceIdType.MESH,
            )
            x_fwd.start()
            send_waits.append(x_fwd)

        x_j = x_ref[...] if h == 0 else comm_x[h]
        part = contribution(x_j, j)

        if h == 0:
            acc[0, :, :] = part
        else:
            pltpu.make_async_remote_copy(
                src_ref=acc.at[h], dst_ref=acc.at[h],
                send_sem=acc_send_sems.at[h], recv_sem=acc_recv_sems.at[h],
                device_id=(left,), device_id_type=pl.DeviceIdType.MESH,
            ).wait_recv()
            acc[h, :, :] = acc[h, :, :] + part

        if h < N_DEV - 1:
            acc_fwd = pltpu.make_async_remote_copy(
                src_ref=acc.at[h], dst_ref=acc.at[h + 1],
                send_sem=acc_send_sems.at[h], recv_sem=acc_recv_sems.at[h + 1],
                device_id=(right,), device_id_type=pl.DeviceIdType.MESH,
            )
            acc_fwd.start()
            send_waits.append(acc_fwd)
        else:
            out_fwd = pltpu.make_async_remote_copy(
                src_ref=acc.at[h], dst_ref=out_ref,
                send_sem=out_send_sem, recv_sem=out_recv_sem,
                device_id=(right,), device_id_type=pl.DeviceIdType.MESH,
            )
            out_fwd.start()
            send_waits.append(out_fwd)

    pltpu.make_async_remote_copy(
        src_ref=acc.at[N_DEV - 1], dst_ref=out_ref,
        send_sem=out_send_sem, recv_sem=out_recv_sem,
        device_id=(left,), device_id_type=pl.DeviceIdType.MESH,
    ).wait_recv()

    for d in send_waits:
        d.wait_send()


def kernel(x, Wq, K_ext, V_ext, Wo):
    my = lax.axis_index("i")
    hstart = H_PER * my
    k_sl = lax.dynamic_slice(K_ext, (0, 0, hstart, 0), (1, K_ROWS, H_PER, DH))[0]
    v_sl = lax.dynamic_slice(V_ext, (0, 0, hstart, 0), (1, K_ROWS, H_PER, DH))[0]
    k_t = jnp.transpose(k_sl, (1, 0, 2))
    v_t = jnp.transpose(v_sl, (1, 0, 2))

    out = pl.pallas_call(
        _body,
        out_shape=jax.ShapeDtypeStruct((SQ_BLK, D_MODEL), jnp.float32),
        in_specs=[pl.BlockSpec(memory_space=pltpu.VMEM)] * 5,
        out_specs=pl.BlockSpec(memory_space=pltpu.VMEM),
        scratch_shapes=[
            pltpu.VMEM((N_DEV, SQ_BLK, D_MODEL), jnp.float32),
            pltpu.VMEM((N_DEV, SQ_BLK, D_MODEL), jnp.float32),
            pltpu.SemaphoreType.DMA((N_DEV,)),
            pltpu.SemaphoreType.DMA((N_DEV,)),
            pltpu.SemaphoreType.DMA((N_DEV,)),
            pltpu.SemaphoreType.DMA((N_DEV,)),
            pltpu.SemaphoreType.DMA,
            pltpu.SemaphoreType.DMA,
        ],
        compiler_params=pltpu.CompilerParams(collective_id=0),
    )(x[0], Wq, k_t, v_t, Wo)
    return out[None]


# device time: 110528 ns/iter; 1.8231x vs baseline; 1.8231x over previous
import jax
import jax.numpy as jnp
from jax import lax
from jax.experimental import pallas as pl
from jax.experimental.pallas import tpu as pltpu

N_DEV = 8
SQ_BLK = 256
D_MODEL = 1024
H_PER = 8
DH = 128
WIN = 512
K_ROWS = 2176
SCALE = 0.08838834764831843
NEG = -1e9


def _body(x_ref, wq_ref, k_ref, v_ref, wo_ref, out_ref,
          comm_x, acc,
          x_send_sems, x_recv_sems, acc_send_sems, acc_recv_sems,
          out_send_sem, out_recv_sem):
    my = lax.axis_index("i")
    right = lax.rem(my + 1, N_DEV)
    left = lax.rem(my + N_DEV - 1, N_DEV)

    barrier = pltpu.get_barrier_semaphore()
    for nbr in (left, right):
        pl.semaphore_signal(barrier, inc=1, device_id=(nbr,),
                            device_id_type=pl.DeviceIdType.MESH)
    pl.semaphore_wait(barrier, 2)

    def contribution(x_j, j):
        q = jnp.dot(x_j, wq_ref[...], preferred_element_type=jnp.float32)
        qb = q.astype(jnp.bfloat16)
        start = pl.multiple_of(jnp.maximum(256 * j - 128, 0), 128)
        rows = lax.broadcasted_iota(jnp.int32, (SQ_BLK, WIN), 0)
        cols = lax.broadcasted_iota(jnp.int32, (SQ_BLK, WIN), 1)
        mask = jnp.abs((rows + 256 * j) - (cols + start)) <= 128
        ctxs = []
        for hh in range(H_PER):
            qh = qb[:, hh * DH:(hh + 1) * DH]
            kh = k_ref[hh, pl.ds(start, WIN), :]
            s = lax.dot_general(qh, kh, (((1,), (1,)), ((), ())),
                                preferred_element_type=jnp.float32)
            s = jnp.where(mask, s * SCALE, NEG)
            m = jnp.max(s, axis=-1, keepdims=True)
            w = jnp.exp(s - m)
            w = (w / jnp.sum(w, axis=-1, keepdims=True)).astype(jnp.bfloat16)
            vh = v_ref[hh, pl.ds(start, WIN), :]
            ctxs.append(jnp.dot(w, vh, preferred_element_type=jnp.float32))
        ctx = jnp.concatenate(ctxs, axis=1).astype(jnp.bfloat16)
        return jnp.dot(ctx, wo_ref[...], preferred_element_type=jnp.float32)

    send_waits = []
    for h in range(N_DEV):
        j = lax.rem(my - h + N_DEV, N_DEV)

        if h == 0:
            x_src = x_ref
        else:
            pltpu.make_async_remote_copy(
                src_ref=comm_x.at[h], dst_ref=comm_x.at[h],
                send_sem=x_send_sems.at[h], recv_sem=x_recv_sems.at[h],
                device_id=(left,), device_id_type=pl.DeviceIdType.MESH,
            ).wait_recv()
            x_src = comm_x.at[h]

        if h < N_DEV - 1:
            x_fwd = pltpu.make_async_remote_copy(
                src_ref=x_src, dst_ref=comm_x.at[h + 1],
                send_sem=x_send_sems.at[h], recv_sem=x_recv_sems.at[h + 1],
                device_id=(right,), device_id_type=pl.DeviceIdType.MESH,
            )
            x_fwd.start()
            send_waits.append(x_fwd)

        x_j = x_ref[...] if h == 0 else comm_x[h]
        part = contribution(x_j, j)

        if h == 0:
            acc[0, :, :] = part.astype(jnp.bfloat16)
        else:
            pltpu.make_async_remote_copy(
                src_ref=acc.at[h], dst_ref=acc.at[h],
                send_sem=acc_send_sems.at[h], recv_sem=acc_recv_sems.at[h],
                device_id=(left,), device_id_type=pl.DeviceIdType.MESH,
            ).wait_recv()
            acc[h, :, :] = (acc[h, :, :].astype(jnp.float32)
                            + part).astype(jnp.bfloat16)

        if h < N_DEV - 1:
            acc_fwd = pltpu.make_async_remote_copy(
                src_ref=acc.at[h], dst_ref=acc.at[h + 1],
                send_sem=acc_send_sems.at[h], recv_sem=acc_recv_sems.at[h + 1],
                device_id=(right,), device_id_type=pl.DeviceIdType.MESH,
            )
            acc_fwd.start()
            send_waits.append(acc_fwd)
        else:
            out_fwd = pltpu.make_async_remote_copy(
                src_ref=acc.at[h], dst_ref=out_ref,
                send_sem=out_send_sem, recv_sem=out_recv_sem,
                device_id=(right,), device_id_type=pl.DeviceIdType.MESH,
            )
            out_fwd.start()
            send_waits.append(out_fwd)

    pltpu.make_async_remote_copy(
        src_ref=acc.at[N_DEV - 1], dst_ref=out_ref,
        send_sem=out_send_sem, recv_sem=out_recv_sem,
        device_id=(left,), device_id_type=pl.DeviceIdType.MESH,
    ).wait_recv()

    for d in send_waits:
        d.wait_send()


def kernel(x, Wq, K_ext, V_ext, Wo):
    my = lax.axis_index("i")
    hstart = H_PER * my
    k_sl = lax.dynamic_slice(K_ext, (0, 0, hstart, 0), (1, K_ROWS, H_PER, DH))[0]
    v_sl = lax.dynamic_slice(V_ext, (0, 0, hstart, 0), (1, K_ROWS, H_PER, DH))[0]
    bf = jnp.bfloat16
    k_t = jnp.transpose(k_sl.astype(bf), (1, 0, 2))
    v_t = jnp.transpose(v_sl.astype(bf), (1, 0, 2))

    out = pl.pallas_call(
        _body,
        out_shape=jax.ShapeDtypeStruct((SQ_BLK, D_MODEL), bf),
        in_specs=[pl.BlockSpec(memory_space=pltpu.VMEM)] * 5,
        out_specs=pl.BlockSpec(memory_space=pltpu.VMEM),
        scratch_shapes=[
            pltpu.VMEM((N_DEV, SQ_BLK, D_MODEL), bf),
            pltpu.VMEM((N_DEV, SQ_BLK, D_MODEL), bf),
            pltpu.SemaphoreType.DMA((N_DEV,)),
            pltpu.SemaphoreType.DMA((N_DEV,)),
            pltpu.SemaphoreType.DMA((N_DEV,)),
            pltpu.SemaphoreType.DMA((N_DEV,)),
            pltpu.SemaphoreType.DMA,
            pltpu.SemaphoreType.DMA,
        ],
        compiler_params=pltpu.CompilerParams(collective_id=0),
    )(x[0].astype(bf), Wq.astype(bf), k_t, v_t, Wo.astype(bf))
    return out[None].astype(jnp.float32)


# device time: 83967 ns/iter; 2.3998x vs baseline; 1.3163x over previous
import jax
import jax.numpy as jnp
from jax import lax
from jax.experimental import pallas as pl
from jax.experimental.pallas import tpu as pltpu

N_DEV = 8
SQ_BLK = 256
D_MODEL = 1024
H_PER = 8
DH = 128
WIN = 512
K_ROWS = 2176
SCALE = 0.08838834764831843
NEG = -1e9
BF = jnp.bfloat16


def _body(x_ref, wq_ref, k_hbm, v_hbm, wo_ref, out_ref,
          kf32, vf32, kb, vb, xl, xr, aacc, bacc,
          kv_sems, xl_s, xl_r, xr_s, xr_r, a_s, a_r, b_s, b_r):
    my = lax.axis_index("i")
    right = lax.rem(my + 1, N_DEV)
    left = lax.rem(my + N_DEV - 1, N_DEV)
    hstart = H_PER * my

    kv_copies = []
    for hh in range(H_PER):
        c = pltpu.make_async_copy(
            k_hbm.at[0, pl.ds(0, K_ROWS), hstart + hh, :],
            kf32.at[hh], kv_sems.at[hh])
        c.start()
        kv_copies.append(c)
        c = pltpu.make_async_copy(
            v_hbm.at[0, pl.ds(0, K_ROWS), hstart + hh, :],
            vf32.at[hh], kv_sems.at[H_PER + hh])
        c.start()
        kv_copies.append(c)

    barrier = pltpu.get_barrier_semaphore()
    for nbr in (left, right):
        pl.semaphore_signal(barrier, inc=1, device_id=(nbr,),
                            device_id_type=pl.DeviceIdType.MESH)
    pl.semaphore_wait(barrier, 2)

    send_waits = []

    def rsend(src, dst, ssem, rsem, dev):
        d = pltpu.make_async_remote_copy(
            src_ref=src, dst_ref=dst, send_sem=ssem, recv_sem=rsem,
            device_id=(dev,), device_id_type=pl.DeviceIdType.MESH)
        d.start()
        send_waits.append(d)

    def rwait(dst, ssem, rsem, src_dev):
        pltpu.make_async_remote_copy(
            src_ref=dst, dst_ref=dst, send_sem=ssem, recv_sem=rsem,
            device_id=(src_dev,), device_id_type=pl.DeviceIdType.MESH,
        ).wait_recv()

    def blk(d):
        return lax.rem(my + d, N_DEV)

    rsend(x_ref, xl.at[1], xl_s.at[1], xl_r.at[1], right)
    rsend(x_ref, xr.at[1], xr_s.at[1], xr_r.at[1], left)
    rwait(xl.at[1], xl_s.at[1], xl_r.at[1], left)
    rsend(xl.at[1], xl.at[2], xl_s.at[2], xl_r.at[2], right)
    rwait(xr.at[1], xr_s.at[1], xr_r.at[1], right)
    rsend(xr.at[1], xr.at[2], xr_s.at[2], xr_r.at[2], left)
    rwait(xl.at[2], xl_s.at[2], xl_r.at[2], left)
    rsend(xl.at[2], xl.at[3], xl_s.at[3], xl_r.at[3], right)
    rwait(xr.at[2], xr_s.at[2], xr_r.at[2], right)
    rsend(xr.at[2], xr.at[3], xr_s.at[3], xr_r.at[3], left)
    rwait(xl.at[3], xl_s.at[3], xl_r.at[3], left)
    rwait(xr.at[3], xr_s.at[3], xr_r.at[3], right)
    rsend(xr.at[3], xr.at[4], xr_s.at[4], xr_r.at[4], left)

    for c in kv_copies:
        c.wait()
    kb[...] = kf32[...].astype(BF)
    vb[...] = vf32[...].astype(BF)

    rows = lax.broadcasted_iota(jnp.int32, (SQ_BLK, WIN), 0)
    cols = lax.broadcasted_iota(jnp.int32, (SQ_BLK, WIN), 1)
    d = rows - cols
    mm_add = jnp.where((d <= 0) & (d >= -256), 0.0, NEG).astype(jnp.float32)
    m0_add = jnp.where(jnp.abs(d) <= 128, 0.0, NEG).astype(jnp.float32)

    def contribution(x_j, j):
        q = jnp.dot(x_j, wq_ref[...], preferred_element_type=jnp.float32)
        qb = q.astype(BF)
        start = pl.multiple_of(jnp.maximum(256 * j - 128, 0), 128)
        madd = jnp.where(j == 0, m0_add, mm_add)
        ctxs = []
        for hh in range(H_PER):
            qh = qb[:, hh * DH:(hh + 1) * DH]
            kh = kb[hh, pl.ds(start, WIN), :]
            s = lax.dot_general(qh, kh, (((1,), (1,)), ((), ())),
                                preferred_element_type=jnp.float32)
            p = jnp.exp(s * SCALE + madd)
            w = (p / jnp.sum(p, axis=-1, keepdims=True)).astype(BF)
            vh = vb[hh, pl.ds(start, WIN), :]
            ctxs.append(jnp.dot(w, vh, preferred_element_type=jnp.float32))
        ctx = jnp.concatenate(ctxs, axis=1).astype(BF)
        return jnp.dot(ctx, wo_ref[...], preferred_element_type=jnp.float32)

    bacc[0, :, :] = contribution(xr[3], blk(3)).astype(BF)
    rsend(bacc.at[0], bacc.at[1], b_s.at[1], b_r.at[1], right)

    rwait(xr.at[4], xr_s.at[4], xr_r.at[4], right)
    aacc[0, :, :] = contribution(xr[4], blk(4)).astype(BF)
    rsend(aacc.at[0], aacc.at[1], a_s.at[1], a_r.at[1], left)


    c2 = contribution(xr[2], blk(2))
    rwait(bacc.at[1], b_s.at[1], b_r.at[1], left)
    bacc[1, :, :] = (bacc[1, :, :].astype(jnp.float32) + c2).astype(BF)
    rsend(bacc.at[1], bacc.at[2], b_s.at[2], b_r.at[2], right)

    c5 = contribution(xl[3], blk(5))
    rwait(aacc.at[1], a_s.at[1], a_r.at[1], right)
    aacc[1, :, :] = (aacc[1, :, :].astype(jnp.float32) + c5).astype(BF)
    rsend(aacc.at[1], aacc.at[2], a_s.at[2], a_r.at[2], left)

    c1 = contribution(xr[1], blk(1))
    rwait(bacc.at[2], b_s.at[2], b_r.at[2], left)
    bacc[2, :, :] = (bacc[2, :, :].astype(jnp.float32) + c1).astype(BF)
    rsend(bacc.at[2], bacc.at[3], b_s.at[3], b_r.at[3], right)

    c6 = contribution(xl[2], blk(6))
    rwait(aacc.at[2], a_s.at[2], a_r.at[2], right)
    aacc[2, :, :] = (aacc[2, :, :].astype(jnp.float32) + c6).astype(BF)
    rsend(aacc.at[2], aacc.at[3], a_s.at[3], a_r.at[3], left)

    c7 = contribution(xl[1], blk(7))
    rwait(aacc.at[3], a_s.at[3], a_r.at[3], right)
    aacc[3, :, :] = (aacc[3, :, :].astype(jnp.float32) + c7).astype(BF)
    rsend(aacc.at[3], aacc.at[4], a_s.at[4], a_r.at[4], left)

    c_own = contribution(x_ref[...], my)
    rwait(bacc.at[3], b_s.at[3], b_r.at[3], left)
    rwait(aacc.at[4], a_s.at[4], a_r.at[4], right)
    out_ref[...] = (aacc[4, :, :].astype(jnp.float32)
                    + bacc[3, :, :].astype(jnp.float32) + c_own).astype(BF)

    for dsc in send_waits:
        dsc.wait_send()


def kernel(x, Wq, K_ext, V_ext, Wo):
    slot = (SQ_BLK, D_MODEL)
    out = pl.pallas_call(
        _body,
        out_shape=jax.ShapeDtypeStruct(slot, BF),
        in_specs=[
            pl.BlockSpec(memory_space=pltpu.VMEM),
            pl.BlockSpec(memory_space=pltpu.VMEM),
            pl.BlockSpec(memory_space=pl.ANY),
            pl.BlockSpec(memory_space=pl.ANY),
            pl.BlockSpec(memory_space=pltpu.VMEM),
        ],
        out_specs=pl.BlockSpec(memory_space=pltpu.VMEM),
        scratch_shapes=[
            pltpu.VMEM((H_PER, K_ROWS, DH), jnp.float32),
            pltpu.VMEM((H_PER, K_ROWS, DH), jnp.float32),
            pltpu.VMEM((H_PER, K_ROWS, DH), BF),
            pltpu.VMEM((H_PER, K_ROWS, DH), BF),
            pltpu.VMEM((4,) + slot, BF),
            pltpu.VMEM((5,) + slot, BF),
            pltpu.VMEM((5,) + slot, BF),
            pltpu.VMEM((4,) + slot, BF),
            pltpu.SemaphoreType.DMA((2 * H_PER,)),
            pltpu.SemaphoreType.DMA((4,)),
            pltpu.SemaphoreType.DMA((4,)),
            pltpu.SemaphoreType.DMA((5,)),
            pltpu.SemaphoreType.DMA((5,)),
            pltpu.SemaphoreType.DMA((5,)),
            pltpu.SemaphoreType.DMA((5,)),
            pltpu.SemaphoreType.DMA((4,)),
            pltpu.SemaphoreType.DMA((4,)),
        ],
        compiler_params=pltpu.CompilerParams(
            collective_id=0, vmem_limit_bytes=100 * 1024 * 1024),
    )(x[0].astype(BF), Wq.astype(BF), K_ext, V_ext, Wo.astype(BF))
    return out[None].astype(jnp.float32)


# device time: 77830 ns/iter; 2.5891x vs baseline; 1.0789x over previous
import jax
import jax.numpy as jnp
from jax import lax
from jax.experimental import pallas as pl
from jax.experimental.pallas import tpu as pltpu

N_DEV = 8
SQ_BLK = 256
D_MODEL = 1024
H_PER = 8
DH = 128
WIN = 512
K_ROWS = 2176
SCALE = 0.08838834764831843
NEG = -1e9
BF = jnp.bfloat16


def _body(x_ref, wq_ref, k_hbm, v_hbm, wo_ref, out_ref,
          kf32, vf32, kb, vb, xb, wqb, wob, xl, xr, aacc, bacc,
          kv_sems, xl_s, xl_r, xr_s, xr_r, a_s, a_r, b_s, b_r):
    my = lax.axis_index("i")
    right = lax.rem(my + 1, N_DEV)
    left = lax.rem(my + N_DEV - 1, N_DEV)
    hstart = H_PER * my

    kv_copies = []
    for hh in range(H_PER):
        c = pltpu.make_async_copy(
            k_hbm.at[0, pl.ds(0, K_ROWS), hstart + hh, :],
            kf32.at[hh], kv_sems.at[hh])
        c.start()
        kv_copies.append(c)
        c = pltpu.make_async_copy(
            v_hbm.at[0, pl.ds(0, K_ROWS), hstart + hh, :],
            vf32.at[hh], kv_sems.at[H_PER + hh])
        c.start()
        kv_copies.append(c)

    xb[...] = x_ref[...].astype(BF)

    barrier = pltpu.get_barrier_semaphore()
    for nbr in (left, right):
        pl.semaphore_signal(barrier, inc=1, device_id=(nbr,),
                            device_id_type=pl.DeviceIdType.MESH)
    pl.semaphore_wait(barrier, 2)

    send_waits = []

    def rsend(src, dst, ssem, rsem, dev):
        d = pltpu.make_async_remote_copy(
            src_ref=src, dst_ref=dst, send_sem=ssem, recv_sem=rsem,
            device_id=(dev,), device_id_type=pl.DeviceIdType.MESH)
        d.start()
        send_waits.append(d)

    def rwait(dst, ssem, rsem, src_dev):
        pltpu.make_async_remote_copy(
            src_ref=dst, dst_ref=dst, send_sem=ssem, recv_sem=rsem,
            device_id=(src_dev,), device_id_type=pl.DeviceIdType.MESH,
        ).wait_recv()

    def blk(d):
        return lax.rem(my + d, N_DEV)

    rsend(xb, xl.at[1], xl_s.at[1], xl_r.at[1], right)
    rsend(xb, xr.at[1], xr_s.at[1], xr_r.at[1], left)
    rwait(xl.at[1], xl_s.at[1], xl_r.at[1], left)
    rsend(xl.at[1], xl.at[2], xl_s.at[2], xl_r.at[2], right)
    rwait(xr.at[1], xr_s.at[1], xr_r.at[1], right)
    rsend(xr.at[1], xr.at[2], xr_s.at[2], xr_r.at[2], left)
    wqb[...] = wq_ref[...].astype(BF)
    wob[...] = wo_ref[...].astype(BF)
    rwait(xl.at[2], xl_s.at[2], xl_r.at[2], left)
    rsend(xl.at[2], xl.at[3], xl_s.at[3], xl_r.at[3], right)
    rwait(xr.at[2], xr_s.at[2], xr_r.at[2], right)
    rsend(xr.at[2], xr.at[3], xr_s.at[3], xr_r.at[3], left)
    rwait(xl.at[3], xl_s.at[3], xl_r.at[3], left)
    rwait(xr.at[3], xr_s.at[3], xr_r.at[3], right)
    rsend(xr.at[3], xr.at[4], xr_s.at[4], xr_r.at[4], left)

    for c in kv_copies:
        c.wait()
    kb[...] = kf32[...].astype(BF)
    vb[...] = vf32[...].astype(BF)

    rows = lax.broadcasted_iota(jnp.int32, (SQ_BLK, WIN), 0)
    cols = lax.broadcasted_iota(jnp.int32, (SQ_BLK, WIN), 1)
    d = rows - cols
    mm_add = jnp.where((d <= 0) & (d >= -256), 0.0, NEG).astype(jnp.float32)
    m0_add = jnp.where(jnp.abs(d) <= 128, 0.0, NEG).astype(jnp.float32)

    def contribution(x_j, j):
        q = jnp.dot(x_j, wqb[...], preferred_element_type=jnp.float32)
        qb = q.astype(BF)
        start = pl.multiple_of(jnp.maximum(256 * j - 128, 0), 128)
        madd = jnp.where(j == 0, m0_add, mm_add)
        ctxs = []
        for hh in range(H_PER):
            qh = qb[:, hh * DH:(hh + 1) * DH]
            kh = kb[hh, pl.ds(start, WIN), :]
            s = lax.dot_general(qh, kh, (((1,), (1,)), ((), ())),
                                preferred_element_type=jnp.float32)
            p = jnp.exp(s * SCALE + madd)
            w = (p / jnp.sum(p, axis=-1, keepdims=True)).astype(BF)
            vh = vb[hh, pl.ds(start, WIN), :]
            ctxs.append(jnp.dot(w, vh, preferred_element_type=jnp.float32))
        ctx = jnp.concatenate(ctxs, axis=1).astype(BF)
        return jnp.dot(ctx, wob[...], preferred_element_type=jnp.float32)

    bacc[0, :, :] = contribution(xr[3], blk(3)).astype(BF)
    rsend(bacc.at[0], bacc.at[1], b_s.at[1], b_r.at[1], right)

    rwait(xr.at[4], xr_s.at[4], xr_r.at[4], right)
    aacc[0, :, :] = contribution(xr[4], blk(4)).astype(BF)
    rsend(aacc.at[0], aacc.at[1], a_s.at[1], a_r.at[1], left)


    c2 = contribution(xr[2], blk(2))
    rwait(bacc.at[1], b_s.at[1], b_r.at[1], left)
    bacc[1, :, :] = (bacc[1, :, :].astype(jnp.float32) + c2).astype(BF)
    rsend(bacc.at[1], bacc.at[2], b_s.at[2], b_r.at[2], right)

    c5 = contribution(xl[3], blk(5))
    rwait(aacc.at[1], a_s.at[1], a_r.at[1], right)
    aacc[1, :, :] = (aacc[1, :, :].astype(jnp.float32) + c5).astype(BF)
    rsend(aacc.at[1], aacc.at[2], a_s.at[2], a_r.at[2], left)

    c1 = contribution(xr[1], blk(1))
    rwait(bacc.at[2], b_s.at[2], b_r.at[2], left)
    bacc[2, :, :] = (bacc[2, :, :].astype(jnp.float32) + c1).astype(BF)
    rsend(bacc.at[2], bacc.at[3], b_s.at[3], b_r.at[3], right)

    c6 = contribution(xl[2], blk(6))
    rwait(aacc.at[2], a_s.at[2], a_r.at[2], right)
    aacc[2, :, :] = (aacc[2, :, :].astype(jnp.float32) + c6).astype(BF)
    rsend(aacc.at[2], aacc.at[3], a_s.at[3], a_r.at[3], left)

    c7 = contribution(xl[1], blk(7))
    rwait(aacc.at[3], a_s.at[3], a_r.at[3], right)
    aacc[3, :, :] = (aacc[3, :, :].astype(jnp.float32) + c7).astype(BF)
    rsend(aacc.at[3], aacc.at[4], a_s.at[4], a_r.at[4], left)

    c_own = contribution(xb[...], my)
    rwait(bacc.at[3], b_s.at[3], b_r.at[3], left)
    rwait(aacc.at[4], a_s.at[4], a_r.at[4], right)
    out_ref[...] = (aacc[4, :, :].astype(jnp.float32)
                    + bacc[3, :, :].astype(jnp.float32) + c_own)

    for dsc in send_waits:
        dsc.wait_send()


def kernel(x, Wq, K_ext, V_ext, Wo):
    slot = (SQ_BLK, D_MODEL)
    out = pl.pallas_call(
        _body,
        out_shape=jax.ShapeDtypeStruct(slot, jnp.float32),
        in_specs=[
            pl.BlockSpec(memory_space=pltpu.VMEM),
            pl.BlockSpec(memory_space=pltpu.VMEM),
            pl.BlockSpec(memory_space=pl.ANY),
            pl.BlockSpec(memory_space=pl.ANY),
            pl.BlockSpec(memory_space=pltpu.VMEM),
        ],
        out_specs=pl.BlockSpec(memory_space=pltpu.VMEM),
        scratch_shapes=[
            pltpu.VMEM((H_PER, K_ROWS, DH), jnp.float32),
            pltpu.VMEM((H_PER, K_ROWS, DH), jnp.float32),
            pltpu.VMEM((H_PER, K_ROWS, DH), BF),
            pltpu.VMEM((H_PER, K_ROWS, DH), BF),
            pltpu.VMEM(slot, BF),
            pltpu.VMEM((D_MODEL, D_MODEL), BF),
            pltpu.VMEM((D_MODEL, D_MODEL), BF),
            pltpu.VMEM((4,) + slot, BF),
            pltpu.VMEM((5,) + slot, BF),
            pltpu.VMEM((5,) + slot, BF),
            pltpu.VMEM((4,) + slot, BF),
            pltpu.SemaphoreType.DMA((2 * H_PER,)),
            pltpu.SemaphoreType.DMA((4,)),
            pltpu.SemaphoreType.DMA((4,)),
            pltpu.SemaphoreType.DMA((5,)),
            pltpu.SemaphoreType.DMA((5,)),
            pltpu.SemaphoreType.DMA((5,)),
            pltpu.SemaphoreType.DMA((5,)),
            pltpu.SemaphoreType.DMA((4,)),
            pltpu.SemaphoreType.DMA((4,)),
        ],
        compiler_params=pltpu.CompilerParams(
            collective_id=0, vmem_limit_bytes=100 * 1024 * 1024),
    )(x[0], Wq, K_ext, V_ext, Wo)
    return out[None]


# device time: 70859 ns/iter; 2.8438x vs baseline; 1.0984x over previous
import jax
import jax.numpy as jnp
from jax import lax
from jax.experimental import pallas as pl
from jax.experimental.pallas import tpu as pltpu

N_DEV = 8
SQ_BLK = 256
HALF = SQ_BLK // 2
D_MODEL = 1024
H_PER = 8
DH = 128
WIN = 512
K_ROWS = 2176
SCALE = 0.08838834764831843
NEG = -1e9
BF = jnp.bfloat16


def _body(x_ref, wq_ref, k_hbm, v_hbm, wo_ref, out_ref,
          kf32, vf32, kb, vb, xb, wqb, wob, xl, xr, aacc, bacc,
          kv_sems, xl_s, xl_r, xr_s, xr_r, a_s, a_r, b_s, b_r):
    my = lax.axis_index("i")
    right = lax.rem(my + 1, N_DEV)
    left = lax.rem(my + N_DEV - 1, N_DEV)
    hstart = H_PER * my

    kv_copies = []
    for hh in range(H_PER):
        c = pltpu.make_async_copy(
            k_hbm.at[0, pl.ds(0, K_ROWS), hstart + hh, :],
            kf32.at[hh], kv_sems.at[hh])
        c.start()
        kv_copies.append(c)
        c = pltpu.make_async_copy(
            v_hbm.at[0, pl.ds(0, K_ROWS), hstart + hh, :],
            vf32.at[hh], kv_sems.at[H_PER + hh])
        c.start()
        kv_copies.append(c)

    xb[...] = x_ref[...].astype(BF)

    barrier = pltpu.get_barrier_semaphore()
    for nbr in (left, right):
        pl.semaphore_signal(barrier, inc=1, device_id=(nbr,),
                            device_id_type=pl.DeviceIdType.MESH)
    pl.semaphore_wait(barrier, 2)

    send_waits = []

    def rsend(src, dst, ssem, rsem, dev):
        dsc = pltpu.make_async_remote_copy(
            src_ref=src, dst_ref=dst, send_sem=ssem, recv_sem=rsem,
            device_id=(dev,), device_id_type=pl.DeviceIdType.MESH)
        dsc.start()
        send_waits.append(dsc)

    def rwait(dst, ssem, rsem, src_dev):
        pltpu.make_async_remote_copy(
            src_ref=dst, dst_ref=dst, send_sem=ssem, recv_sem=rsem,
            device_id=(src_dev,), device_id_type=pl.DeviceIdType.MESH,
        ).wait_recv()

    def blk(d):
        return lax.rem(my + d, N_DEV)

    def hs(hi):
        return pl.ds(hi * HALF, HALF)

    def sidx(t, hi):
        return 2 * t + hi

    for hi in range(2):
        rsend(xb.at[hs(hi)], xl.at[1, hs(hi)],
              xl_s.at[sidx(1, hi)], xl_r.at[sidx(1, hi)], right)
        rsend(xb.at[hs(hi)], xr.at[1, hs(hi)],
              xr_s.at[sidx(1, hi)], xr_r.at[sidx(1, hi)], left)
    for t in (1, 2):
        for hi in range(2):
            rwait(xl.at[t, hs(hi)], xl_s.at[sidx(t, hi)],
                  xl_r.at[sidx(t, hi)], left)
            rsend(xl.at[t, hs(hi)], xl.at[t + 1, hs(hi)],
                  xl_s.at[sidx(t + 1, hi)], xl_r.at[sidx(t + 1, hi)], right)
            rwait(xr.at[t, hs(hi)], xr_s.at[sidx(t, hi)],
                  xr_r.at[sidx(t, hi)], right)
            rsend(xr.at[t, hs(hi)], xr.at[t + 1, hs(hi)],
                  xr_s.at[sidx(t + 1, hi)], xr_r.at[sidx(t + 1, hi)], left)
        if t == 1:
            wqb[...] = wq_ref[...].astype(BF)
            wob[...] = wo_ref[...].astype(BF)
    for hi in range(2):
        rwait(xl.at[3, hs(hi)], xl_s.at[sidx(3, hi)],
              xl_r.at[sidx(3, hi)], left)
        rwait(xr.at[3, hs(hi)], xr_s.at[sidx(3, hi)],
              xr_r.at[sidx(3, hi)], right)
        rsend(xr.at[3, hs(hi)], xr.at[4, hs(hi)],
              xr_s.at[sidx(4, hi)], xr_r.at[sidx(4, hi)], left)

    for c in kv_copies:
        c.wait()
    kb[...] = kf32[...].astype(BF)
    vb[...] = vf32[...].astype(BF)

    rows = lax.broadcasted_iota(jnp.int32, (SQ_BLK, WIN), 0)
    cols = lax.broadcasted_iota(jnp.int32, (SQ_BLK, WIN), 1)
    d = rows - cols
    mm_add = jnp.where((d <= 0) & (d >= -256), 0.0, NEG).astype(jnp.float32)
    m0_add = jnp.where(jnp.abs(d) <= 128, 0.0, NEG).astype(jnp.float32)

    def contribution(x_j, j):
        q = jnp.dot(x_j, wqb[...], preferred_element_type=jnp.float32)
        qb = q.astype(BF)
        start = pl.multiple_of(jnp.maximum(256 * j - 128, 0), 128)
        madd = jnp.where(j == 0, m0_add, mm_add)
        ctxs = []
        for hh in range(H_PER):
            qh = qb[:, hh * DH:(hh + 1) * DH]
            kh = kb[hh, pl.ds(start, WIN), :]
            s = lax.dot_general(qh, kh, (((1,), (1,)), ((), ())),
                                preferred_element_type=jnp.float32)
            p = jnp.exp(s * SCALE + madd)
            w = (p / jnp.sum(p, axis=-1, keepdims=True)).astype(BF)
            vh = vb[hh, pl.ds(start, WIN), :]
            ctxs.append(jnp.dot(w, vh, preferred_element_type=jnp.float32))
        ctx = jnp.concatenate(ctxs, axis=1).astype(BF)
        return jnp.dot(ctx, wob[...], preferred_element_type=jnp.float32)

    def chain_stage(acc, s, ssems, rsems, from_dev, to_dev, c):
        for hi in range(2):
            rwait(acc.at[s, hs(hi)], ssems.at[sidx(s, hi)],
                  rsems.at[sidx(s, hi)], from_dev)
            lo, hi_ = hi * HALF, (hi + 1) * HALF
            acc[s, lo:hi_, :] = (acc[s, lo:hi_, :].astype(jnp.float32)
                                 + c[lo:hi_, :]).astype(BF)
            rsend(acc.at[s, hs(hi)], acc.at[s + 1, hs(hi)],
                  ssems.at[sidx(s + 1, hi)], rsems.at[sidx(s + 1, hi)],
                  to_dev)

    bacc[0, :, :] = contribution(xr[3], blk(3)).astype(BF)
    for hi in range(2):
        rsend(bacc.at[0, hs(hi)], bacc.at[1, hs(hi)],
              b_s.at[sidx(1, hi)], b_r.at[sidx(1, hi)], right)

    for hi in range(2):
        rwait(xr.at[4, hs(hi)], xr_s.at[sidx(4, hi)],
              xr_r.at[sidx(4, hi)], right)
    aacc[0, :, :] = contribution(xr[4], blk(4)).astype(BF)
    for hi in range(2):
        rsend(aacc.at[0, hs(hi)], aacc.at[1, hs(hi)],
              a_s.at[sidx(1, hi)], a_r.at[sidx(1, hi)], left)


    c2 = contribution(xr[2], blk(2))
    chain_stage(bacc, 1, b_s, b_r, left, right, c2)

    c5 = contribution(xl[3], blk(5))
    chain_stage(aacc, 1, a_s, a_r, right, left, c5)

    c1 = contribution(xr[1], blk(1))
    chain_stage(bacc, 2, b_s, b_r, left, right, c1)

    c6 = contribution(xl[2], blk(6))
    chain_stage(aacc, 2, a_s, a_r, right, left, c6)

    c7 = contribution(xl[1], blk(7))
    chain_stage(aacc, 3, a_s, a_r, right, left, c7)

    c_own = contribution(xb[...], my)
    for hi in range(2):
        rwait(bacc.at[3, hs(hi)], b_s.at[sidx(3, hi)],
              b_r.at[sidx(3, hi)], left)
        rwait(aacc.at[4, hs(hi)], a_s.at[sidx(4, hi)],
              a_r.at[sidx(4, hi)], right)
    out_ref[...] = (aacc[4, :, :].astype(jnp.float32)
                    + bacc[3, :, :].astype(jnp.float32) + c_own)

    for dsc in send_waits:
        dsc.wait_send()


def kernel(x, Wq, K_ext, V_ext, Wo):
    slot = (SQ_BLK, D_MODEL)
    out = pl.pallas_call(
        _body,
        out_shape=jax.ShapeDtypeStruct(slot, jnp.float32),
        in_specs=[
            pl.BlockSpec(memory_space=pltpu.VMEM),
            pl.BlockSpec(memory_space=pltpu.VMEM),
            pl.BlockSpec(memory_space=pl.ANY),
            pl.BlockSpec(memory_space=pl.ANY),
            pl.BlockSpec(memory_space=pltpu.VMEM),
        ],
        out_specs=pl.BlockSpec(memory_space=pltpu.VMEM),
        scratch_shapes=[
            pltpu.VMEM((H_PER, K_ROWS, DH), jnp.float32),
            pltpu.VMEM((H_PER, K_ROWS, DH), jnp.float32),
            pltpu.VMEM((H_PER, K_ROWS, DH), BF),
            pltpu.VMEM((H_PER, K_ROWS, DH), BF),
            pltpu.VMEM(slot, BF),
            pltpu.VMEM((D_MODEL, D_MODEL), BF),
            pltpu.VMEM((D_MODEL, D_MODEL), BF),
            pltpu.VMEM((4,) + slot, BF),
            pltpu.VMEM((5,) + slot, BF),
            pltpu.VMEM((5,) + slot, BF),
            pltpu.VMEM((4,) + slot, BF),
            pltpu.SemaphoreType.DMA((2 * H_PER,)),
            pltpu.SemaphoreType.DMA((8,)),
            pltpu.SemaphoreType.DMA((8,)),
            pltpu.SemaphoreType.DMA((10,)),
            pltpu.SemaphoreType.DMA((10,)),
            pltpu.SemaphoreType.DMA((10,)),
            pltpu.SemaphoreType.DMA((10,)),
            pltpu.SemaphoreType.DMA((8,)),
            pltpu.SemaphoreType.DMA((8,)),
        ],
        compiler_params=pltpu.CompilerParams(
            collective_id=0, vmem_limit_bytes=100 * 1024 * 1024),
    )(x[0], Wq, K_ext, V_ext, Wo)
    return out[None]


# device time: 66410 ns/iter; 3.0343x vs baseline; 1.0670x over previous
import jax
import jax.numpy as jnp
from jax import lax
from jax.experimental import pallas as pl
from jax.experimental.pallas import tpu as pltpu

N_DEV = 8
SQ_BLK = 256
HALF = SQ_BLK // 2
D_MODEL = 1024
H_PER = 8
DH = 128
WIN = 512
K_ROWS = 2176
SCALE = 0.08838834764831843
NEG = -1e9
BF = jnp.bfloat16


def _body(x_ref, wq_ref, k_hbm, v_hbm, wo_ref, out_ref,
          kf32, vf32, kb, vb, xb, wqb, wob, xl, xr, aacc, bacc,
          kv_sems, xl_s, xl_r, xr_s, xr_r, a_s, a_r, b_s, b_r):
    my = lax.axis_index("i")
    right = lax.rem(my + 1, N_DEV)
    left = lax.rem(my + N_DEV - 1, N_DEV)
    hstart = H_PER * my

    kv_copies = []
    for hh in range(H_PER):
        c = pltpu.make_async_copy(
            k_hbm.at[0, pl.ds(0, K_ROWS), hstart + hh, :],
            kf32.at[hh], kv_sems.at[hh])
        c.start()
        kv_copies.append(c)
        c = pltpu.make_async_copy(
            v_hbm.at[0, pl.ds(0, K_ROWS), hstart + hh, :],
            vf32.at[hh], kv_sems.at[H_PER + hh])
        c.start()
        kv_copies.append(c)

    xb[...] = x_ref[...].astype(BF)

    barrier = pltpu.get_barrier_semaphore()
    for nbr in (left, right):
        pl.semaphore_signal(barrier, inc=1, device_id=(nbr,),
                            device_id_type=pl.DeviceIdType.MESH)
    pl.semaphore_wait(barrier, 2)

    send_waits = []

    def rsend(src, dst, ssem, rsem, dev):
        dsc = pltpu.make_async_remote_copy(
            src_ref=src, dst_ref=dst, send_sem=ssem, recv_sem=rsem,
            device_id=(dev,), device_id_type=pl.DeviceIdType.MESH)
        dsc.start()
        send_waits.append(dsc)

    def rwait(dst, ssem, rsem, src_dev):
        pltpu.make_async_remote_copy(
            src_ref=dst, dst_ref=dst, send_sem=ssem, recv_sem=rsem,
            device_id=(src_dev,), device_id_type=pl.DeviceIdType.MESH,
        ).wait_recv()

    def blk(d):
        return lax.rem(my + d, N_DEV)

    def hs(hi):
        return pl.ds(hi * HALF, HALF)

    def sidx(t, hi):
        return 2 * t + hi

    for hi in range(2):
        rsend(xb.at[hs(hi)], xl.at[1, hs(hi)],
              xl_s.at[sidx(1, hi)], xl_r.at[sidx(1, hi)], right)
        rsend(xb.at[hs(hi)], xr.at[1, hs(hi)],
              xr_s.at[sidx(1, hi)], xr_r.at[sidx(1, hi)], left)
    for t in (1, 2):
        for hi in range(2):
            rwait(xl.at[t, hs(hi)], xl_s.at[sidx(t, hi)],
                  xl_r.at[sidx(t, hi)], left)
            rsend(xl.at[t, hs(hi)], xl.at[t + 1, hs(hi)],
                  xl_s.at[sidx(t + 1, hi)], xl_r.at[sidx(t + 1, hi)], right)
            rwait(xr.at[t, hs(hi)], xr_s.at[sidx(t, hi)],
                  xr_r.at[sidx(t, hi)], right)
            rsend(xr.at[t, hs(hi)], xr.at[t + 1, hs(hi)],
                  xr_s.at[sidx(t + 1, hi)], xr_r.at[sidx(t + 1, hi)], left)
        if t == 1:
            wqb[...] = wq_ref[...].astype(BF)
            wob[...] = wo_ref[...].astype(BF)
    for hi in range(2):
        rwait(xl.at[3, hs(hi)], xl_s.at[sidx(3, hi)],
              xl_r.at[sidx(3, hi)], left)
        rwait(xr.at[3, hs(hi)], xr_s.at[sidx(3, hi)],
              xr_r.at[sidx(3, hi)], right)
        rsend(xr.at[3, hs(hi)], xr.at[4, hs(hi)],
              xr_s.at[sidx(4, hi)], xr_r.at[sidx(4, hi)], left)

    for c in kv_copies:
        c.wait()
    kb[...] = kf32[...].astype(BF)
    vb[...] = vf32[...].astype(BF)

    rows = lax.broadcasted_iota(jnp.int32, (SQ_BLK, WIN), 0)
    cols = lax.broadcasted_iota(jnp.int32, (SQ_BLK, WIN), 1)
    d = rows - cols
    mm_add = jnp.where((d <= 0) & (d >= -256), 0.0, NEG).astype(jnp.float32)
    m0_add = jnp.where(jnp.abs(d) <= 128, 0.0, NEG).astype(jnp.float32)

    def contribution(x_j, j):
        q = jnp.dot(x_j, wqb[...], preferred_element_type=jnp.float32)
        qb = (q * SCALE).astype(BF)
        start = pl.multiple_of(jnp.maximum(256 * j - 128, 0), 128)
        madd = jnp.where(j == 0, m0_add, mm_add)
        ctxs = []
        for hh in range(H_PER):
            qh = qb[:, hh * DH:(hh + 1) * DH]
            kh = kb[hh, pl.ds(start, WIN), :]
            s = lax.dot_general(qh, kh, (((1,), (1,)), ((), ())),
                                preferred_element_type=jnp.float32)
            p = jnp.exp(s + madd)
            vh = vb[hh, pl.ds(start, WIN), :]
            num = jnp.dot(p.astype(BF), vh,
                          preferred_element_type=jnp.float32)
            ctxs.append(num / jnp.sum(p, axis=-1, keepdims=True))
        ctx = jnp.concatenate(ctxs, axis=1).astype(BF)
        return jnp.dot(ctx, wob[...], preferred_element_type=jnp.float32)

    def chain_stage(acc, s, ssems, rsems, from_dev, to_dev, c):
        for hi in range(2):
            rwait(acc.at[s, hs(hi)], ssems.at[sidx(s, hi)],
                  rsems.at[sidx(s, hi)], from_dev)
            lo, hi_ = hi * HALF, (hi + 1) * HALF
            acc[s, lo:hi_, :] = (acc[s, lo:hi_, :].astype(jnp.float32)
                                 + c[lo:hi_, :]).astype(BF)
            rsend(acc.at[s, hs(hi)], acc.at[s + 1, hs(hi)],
                  ssems.at[sidx(s + 1, hi)], rsems.at[sidx(s + 1, hi)],
                  to_dev)

    bacc[0, :, :] = contribution(xr[3], blk(3)).astype(BF)
    for hi in range(2):
        rsend(bacc.at[0, hs(hi)], bacc.at[1, hs(hi)],
              b_s.at[sidx(1, hi)], b_r.at[sidx(1, hi)], right)

    for hi in range(2):
        rwait(xr.at[4, hs(hi)], xr_s.at[sidx(4, hi)],
              xr_r.at[sidx(4, hi)], right)
    aacc[0, :, :] = contribution(xr[4], blk(4)).astype(BF)
    for hi in range(2):
        rsend(aacc.at[0, hs(hi)], aacc.at[1, hs(hi)],
              a_s.at[sidx(1, hi)], a_r.at[sidx(1, hi)], left)


    c2 = contribution(xr[2], blk(2))
    chain_stage(bacc, 1, b_s, b_r, left, right, c2)

    c5 = contribution(xl[3], blk(5))
    chain_stage(aacc, 1, a_s, a_r, right, left, c5)

    c1 = contribution(xr[1], blk(1))
    chain_stage(bacc, 2, b_s, b_r, left, right, c1)

    c6 = contribution(xl[2], blk(6))
    chain_stage(aacc, 2, a_s, a_r, right, left, c6)

    c7 = contribution(xl[1], blk(7))
    chain_stage(aacc, 3, a_s, a_r, right, left, c7)

    c_own = contribution(xb[...], my)
    for hi in range(2):
        rwait(bacc.at[3, hs(hi)], b_s.at[sidx(3, hi)],
              b_r.at[sidx(3, hi)], left)
        rwait(aacc.at[4, hs(hi)], a_s.at[sidx(4, hi)],
              a_r.at[sidx(4, hi)], right)
    out_ref[...] = (aacc[4, :, :].astype(jnp.float32)
                    + bacc[3, :, :].astype(jnp.float32) + c_own)

    for dsc in send_waits:
        dsc.wait_send()


def kernel(x, Wq, K_ext, V_ext, Wo):
    slot = (SQ_BLK, D_MODEL)
    out = pl.pallas_call(
        _body,
        out_shape=jax.ShapeDtypeStruct(slot, jnp.float32),
        in_specs=[
            pl.BlockSpec(memory_space=pltpu.VMEM),
            pl.BlockSpec(memory_space=pltpu.VMEM),
            pl.BlockSpec(memory_space=pl.ANY),
            pl.BlockSpec(memory_space=pl.ANY),
            pl.BlockSpec(memory_space=pltpu.VMEM),
        ],
        out_specs=pl.BlockSpec(memory_space=pltpu.VMEM),
        scratch_shapes=[
            pltpu.VMEM((H_PER, K_ROWS, DH), jnp.float32),
            pltpu.VMEM((H_PER, K_ROWS, DH), jnp.float32),
            pltpu.VMEM((H_PER, K_ROWS, DH), BF),
            pltpu.VMEM((H_PER, K_ROWS, DH), BF),
            pltpu.VMEM(slot, BF),
            pltpu.VMEM((D_MODEL, D_MODEL), BF),
            pltpu.VMEM((D_MODEL, D_MODEL), BF),
            pltpu.VMEM((4,) + slot, BF),
            pltpu.VMEM((5,) + slot, BF),
            pltpu.VMEM((5,) + slot, BF),
            pltpu.VMEM((4,) + slot, BF),
            pltpu.SemaphoreType.DMA((2 * H_PER,)),
            pltpu.SemaphoreType.DMA((8,)),
            pltpu.SemaphoreType.DMA((8,)),
            pltpu.SemaphoreType.DMA((10,)),
            pltpu.SemaphoreType.DMA((10,)),
            pltpu.SemaphoreType.DMA((10,)),
            pltpu.SemaphoreType.DMA((10,)),
            pltpu.SemaphoreType.DMA((8,)),
            pltpu.SemaphoreType.DMA((8,)),
        ],
        compiler_params=pltpu.CompilerParams(
            collective_id=0, vmem_limit_bytes=100 * 1024 * 1024),
    )(x[0], Wq, K_ext, V_ext, Wo)
    return out[None]
